# baseline (speedup 1.0000x reference)
import functools

import jax
import jax.numpy as jnp
from jax import lax
from jax.experimental import pallas as pl
from jax.experimental.pallas import tpu as pltpu
from jax.experimental.pallas import tpu_sc as plsc

_CH = 128
_L = 16


def _make_gather(N, V2, D, num_cores, num_subcores):
    NW = num_cores * num_subcores
    per_w = N // NW
    CH, L = _CH, _L
    n_ch = per_w // CH
    n_pair = n_ch // 2
    assert n_ch % 2 == 0 and n_pair >= 2

    mesh = plsc.VectorSubcoreMesh(core_axis_name="c", subcore_axis_name="s")

    scratch = [
        pltpu.VMEM((n_ch, CH), jnp.int32),
        pltpu.VMEM((CH,), jnp.int32),
        pltpu.VMEM((CH,), jnp.int32),
        pltpu.VMEM((CH, 2 * D), jnp.float32),
        pltpu.VMEM((CH, 2 * D), jnp.float32),
        pltpu.VMEM((CH, D), jnp.float32),
        pltpu.VMEM((CH, D), jnp.float32),
        pltpu.SemaphoreType.DMA,
        pltpu.SemaphoreType.DMA,
        pltpu.SemaphoreType.DMA,
        pltpu.SemaphoreType.DMA,
    ]

    @functools.partial(
        pl.kernel,
        mesh=mesh,
        out_type=jax.ShapeDtypeStruct((N, D), jnp.float32),
        scratch_types=scratch,
    )
    def gather_kernel(idx_hbm, table_hbm, out_hbm,
                      idx_v, ph0, ph1, buf0, buf1, ob0, ob1,
                      g0, g1, w0, w1):
        phs = (ph0, ph1)
        bufs = (buf0, buf1)
        obs = (ob0, ob1)
        gsem = (g0, g1)
        wsem = (w0, w1)
        wid = lax.axis_index("s") * num_cores + lax.axis_index("c")
        cbase = wid * n_ch

        pltpu.sync_copy(idx_hbm.at[pl.ds(cbase, n_ch)], idx_v)

        def fire_gather(i, b):
            for k in range(CH // L):
                sl = pl.ds(k * L, L)
                phs[b][sl] = lax.shift_right_logical(idx_v[i, sl], 1)
            pltpu.async_copy(table_hbm.at[phs[b]], bufs[b], gsem[b])

        def gather_wait(b):
            pltpu.make_async_copy(table_hbm.at[phs[b]], bufs[b], gsem[b]).wait()

        def select(i, b):
            def grp(r, carry):
                base = lax.mul(r, L)
                tokv = idx_v[i, pl.ds(base, L)]
                for jj in range(L):
                    j = base + jj
                    off = lax.mul(lax.bitwise_and(tokv[jj], 1), D)
                    for k in range(D // L):
                        obs[b][j, pl.ds(k * L, L)] = bufs[b][
                            j, pl.ds(off + k * L, L)
                        ]
                return carry

            lax.fori_loop(0, CH // L, grp, 0)

        def writeback(i, b):
            pltpu.async_copy(
                obs[b], out_hbm.at[pl.ds((cbase + i) * CH, CH)], wsem[b]
            )

        def writeback_wait(i, b):
            pltpu.make_async_copy(
                obs[b], out_hbm.at[pl.ds((cbase + i) * CH, CH)], wsem[b]
            ).wait()

        def step(i, b, first, last):
            if not last:
                fire_gather(i + 1, 1 - b)
            gather_wait(b)
            if not first:
                writeback_wait(i - 2, b)
            select(i, b)
            writeback(i, b)

        fire_gather(0, 0)
        step(0, 0, first=True, last=False)
        step(1, 1, first=True, last=False)

        def body(g, carry):
            i = 2 * g
            step(i, 0, first=False, last=False)
            step(i + 1, 1, first=False, last=False)
            return carry

        lax.fori_loop(1, n_pair - 1, body, 0)

        i = n_ch - 2
        step(i, 0, first=False, last=False)
        step(i + 1, 1, first=False, last=True)
        writeback_wait(n_ch - 2, 0)
        writeback_wait(n_ch - 1, 1)

    return gather_kernel


def kernel(token_ids, embeddings):
    B, H = token_ids.shape
    V, D = embeddings.shape
    flat = token_ids.reshape(-1).astype(jnp.int32)
    N = flat.shape[0]
    info = plsc.get_sparse_core_info()
    table2 = embeddings.reshape(V // 2, 2 * D)
    idx2d = flat.reshape(N // _CH, _CH)
    out = _make_gather(N, V // 2, D, info.num_cores, info.num_subcores)(
        idx2d, table2
    )
    return out.reshape(B, H, D)


# speedup vs baseline: 1.3166x; 1.3166x over previous
import functools

import jax
import jax.numpy as jnp
from jax import lax
from jax.experimental import pallas as pl
from jax.experimental.pallas import tpu as pltpu
from jax.experimental.pallas import tpu_sc as plsc

_CH = 128
_NBUF = 10
_P = 8


def _make_gather(N, V, D, num_cores, num_subcores):
    NW = num_cores * num_subcores
    per_w = N // NW
    CH, NBUF, P = _CH, _NBUF, _P
    n_ch = per_w // CH
    n_outer = n_ch // NBUF
    assert n_ch % NBUF == 0 and n_outer >= 3

    mesh = plsc.VectorSubcoreMesh(core_axis_name="c", subcore_axis_name="s")

    scratch = [
        pltpu.VMEM((n_ch, CH), jnp.int32),
        pltpu.VMEM((NBUF, CH, D), jnp.float32),
    ]
    scratch += [pltpu.SemaphoreType.DMA] * (2 * NBUF)

    @functools.partial(
        pl.kernel,
        mesh=mesh,
        out_type=jax.ShapeDtypeStruct((N, D), jnp.float32),
        scratch_types=scratch,
        compiler_params=pltpu.CompilerParams(use_tc_tiling_on_sc=False),
    )
    def gather_kernel(idx_hbm, table_hbm, out_hbm, idx_v, bufs, *sems):
        gsem = sems[:NBUF]
        wsem = sems[NBUF:]
        wid = lax.axis_index("s") * num_cores + lax.axis_index("c")
        cbase = wid * n_ch

        pltpu.sync_copy(idx_hbm.at[pl.ds(cbase, n_ch)], idx_v)

        def gather(i, b):
            pltpu.async_copy(table_hbm.at[idx_v.at[i]], bufs.at[b], gsem[b])

        def gather_wait(i, b):
            pltpu.make_async_copy(
                table_hbm.at[idx_v.at[i]], bufs.at[b], gsem[b]
            ).wait()

        def writeback(i, b):
            pltpu.async_copy(
                bufs.at[b], out_hbm.at[pl.ds((cbase + i) * CH, CH)], wsem[b]
            )

        def writeback_wait(i, b):
            pltpu.make_async_copy(
                bufs.at[b], out_hbm.at[pl.ds((cbase + i) * CH, CH)], wsem[b]
            ).wait()

        def step(i, b, wb_wait, fire):
            gather_wait(i, b)
            writeback(i, b)
            if wb_wait:
                writeback_wait(i - 2, (b - 2) % NBUF)
            if fire:
                gather(i + P, (b + P) % NBUF)

        for b in range(P):
            gather(b, b)
        for b in range(NBUF):
            step(b, b, wb_wait=(b >= 2), fire=True)

        def body(g, carry):
            i0 = g * NBUF
            for b in range(NBUF):
                step(i0 + b, b, wb_wait=True, fire=True)
            return carry

        lax.fori_loop(1, n_outer - 1, body, 0)

        i0 = (n_outer - 1) * NBUF
        for b in range(NBUF):
            step(i0 + b, b, wb_wait=True, fire=(i0 + b + P < n_ch))
        writeback_wait(n_ch - 2, (NBUF - 2) % NBUF)
        writeback_wait(n_ch - 1, NBUF - 1)

    return gather_kernel


def kernel(token_ids, embeddings):
    B, H = token_ids.shape
    V, D = embeddings.shape
    flat = token_ids.T.reshape(-1).astype(jnp.int32)
    N = flat.shape[0]
    info = plsc.get_sparse_core_info()
    idx2d = flat.reshape(N // _CH, _CH)
    out = _make_gather(N, V, D, info.num_cores, info.num_subcores)(
        idx2d, embeddings
    )
    return out.reshape(H, B, D).swapaxes(0, 1)
